# baseline (speedup 1.0000x reference)
import jax
import jax.numpy as jnp
from jax import lax
from jax.experimental import pallas as pl
from jax.experimental.pallas import tpu as pltpu
from jax.experimental.pallas import tpu_sc as plsc

N = 64
HH = 512
ROWW = 1536
G = 32
NP = G * G
NUM_MASKED = 768
NC, NS = 2, 16
NW = NC * NS


def _sc_mask_body(idx_hbm, mask_hbm, idx_v, mask_v):
    wid = lax.axis_index("s") * NC + lax.axis_index("c")
    zeros16 = jnp.zeros((16,), jnp.int32)
    ones16 = jnp.ones((16,), jnp.int32)

    for s in range(2):
        b = 2 * wid + s
        pltpu.sync_copy(
            idx_hbm.at[pl.ds(8 * (b // 8), 8),
                       pl.ds(NUM_MASKED, NP - NUM_MASKED)],
            idx_v)
        rb = b % 8
        for r in range(G):
            mask_v[r, pl.ds(0, 16)] = zeros16
            mask_v[r, pl.ds(16, 16)] = zeros16
        for j in range(16):
            iv = idx_v[rb, pl.ds(j * 16, 16)]
            plsc.store_scatter(mask_v, [iv >> 5, iv & (G - 1)], ones16)
        pltpu.sync_copy(mask_v, mask_hbm.at[b])


HB = 128


def _tc_select_body(mask_ref, x_ref, tok_ref, rrows_ref, rcols_ref, out_ref):
    u = mask_ref[0].astype(jnp.bfloat16)
    mrows = jax.lax.dot_general(
        rrows_ref[...], u, (((1,), (0,)), ((), ())),
        preferred_element_type=jnp.float32)
    m = jax.lax.dot_general(
        mrows.astype(jnp.bfloat16), rcols_ref[...], (((1,), (0,)), ((), ())),
        preferred_element_type=jnp.float32)
    out_ref[0] = jnp.where(m > 0.5, x_ref[0], tok_ref[...])


def kernel(x, masked_token, shuffle_indices):
    idx2 = shuffle_indices.astype(jnp.int32)

    sc_mesh = plsc.VectorSubcoreMesh(core_axis_name="c", subcore_axis_name="s",
                                     num_cores=NC, num_subcores=NS)
    sc_mask = pl.kernel(
        _sc_mask_body,
        out_type=jax.ShapeDtypeStruct((N, G, G), jnp.int32),
        mesh=sc_mesh,
        compiler_params=pltpu.CompilerParams(needs_layout_passes=False),
        scratch_types=[
            pltpu.VMEM((8, NP - NUM_MASKED), jnp.int32),
            pltpu.VMEM((G, G), jnp.int32),
        ],
    )
    mask = sc_mask(idx2)

    tok_full = jnp.tile(masked_token.reshape(16, 48), (G, G))
    rrows = (jnp.arange(HH, dtype=jnp.int32)[:, None] // 16
             == jnp.arange(G, dtype=jnp.int32)[None, :]).astype(jnp.bfloat16)
    rcols = (jnp.arange(ROWW, dtype=jnp.int32)[None, :] // 48
             == jnp.arange(G, dtype=jnp.int32)[:, None]).astype(jnp.bfloat16)

    NQ = 8
    QB = N // NQ
    tc_select = pl.pallas_call(
        _tc_select_body,
        grid=(QB, HH // HB),
        in_specs=[
            pl.BlockSpec((1, G, G), lambda b, h: (b, 0, 0)),
            pl.BlockSpec((1, HB, ROWW), lambda b, h: (b, h, 0)),
            pl.BlockSpec((HB, ROWW), lambda b, h: (h, 0)),
            pl.BlockSpec((HB, G), lambda b, h: (h, 0)),
            pl.BlockSpec((G, ROWW), lambda b, h: (0, 0)),
        ],
        out_specs=pl.BlockSpec((1, HB, ROWW), lambda b, h: (b, h, 0)),
        out_shape=jax.ShapeDtypeStruct((QB, HH, ROWW), jnp.float32),
    )
    outs = []
    for q in range(NQ):
        xq = x[q * QB:(q + 1) * QB].reshape(QB, HH, ROWW)
        mq = lax.slice_in_dim(mask, q * QB, (q + 1) * QB, axis=0)
        oq = tc_select(mq, xq, tok_full, rrows, rcols)
        outs.append(oq.reshape(QB, HH, HH, 3))
    return jnp.concatenate(outs, axis=0)


# speedup vs baseline: 1.3714x; 1.3714x over previous
import jax
import jax.numpy as jnp
from jax import lax
from jax.experimental import pallas as pl
from jax.experimental.pallas import tpu as pltpu
from jax.experimental.pallas import tpu_sc as plsc

N = 64
HH = 512
ROWW = 1536
G = 32
NP = G * G
NUM_MASKED = 768
NC, NS = 2, 16
NW = NC * NS


def _sc_mask_body(idx_hbm, mask_hbm, idx_v, mask_v):
    wid = lax.axis_index("s") * NC + lax.axis_index("c")
    zeros16 = jnp.zeros((16,), jnp.int32)
    ones16 = jnp.ones((16,), jnp.int32)

    for s in range(2):
        b = 2 * wid + s
        pltpu.sync_copy(
            idx_hbm.at[pl.ds(8 * (b // 8), 8),
                       pl.ds(NUM_MASKED, NP - NUM_MASKED)],
            idx_v)
        rb = b % 8
        for r in range(G):
            mask_v[r, pl.ds(0, 16)] = zeros16
            mask_v[r, pl.ds(16, 16)] = zeros16
        for j in range(16):
            iv = idx_v[rb, pl.ds(j * 16, 16)]
            plsc.store_scatter(mask_v, [iv >> 5, iv & (G - 1)], ones16)
        pltpu.sync_copy(mask_v, mask_hbm.at[b])


HB = 128


def _tc_select_body(mask_ref, x_ref, tok_ref, rrows_ref, rcols_ref, out_ref):
    u = mask_ref[0].astype(jnp.bfloat16)
    mrows = jax.lax.dot_general(
        rrows_ref[...], u, (((1,), (0,)), ((), ())),
        preferred_element_type=jnp.float32)
    m = jax.lax.dot_general(
        mrows.astype(jnp.bfloat16), rcols_ref[...], (((1,), (0,)), ((), ())),
        preferred_element_type=jnp.float32)
    out_ref[0] = jnp.where(m > 0.5, x_ref[0], tok_ref[...])


def kernel(x, masked_token, shuffle_indices):
    idx2 = shuffle_indices.astype(jnp.int32)

    sc_mesh = plsc.VectorSubcoreMesh(core_axis_name="c", subcore_axis_name="s",
                                     num_cores=NC, num_subcores=NS)
    sc_mask = pl.kernel(
        _sc_mask_body,
        out_type=jax.ShapeDtypeStruct((N, G, G), jnp.int32),
        mesh=sc_mesh,
        compiler_params=pltpu.CompilerParams(needs_layout_passes=False),
        scratch_types=[
            pltpu.VMEM((8, NP - NUM_MASKED), jnp.int32),
            pltpu.VMEM((G, G), jnp.int32),
        ],
    )
    mask = sc_mask(idx2)

    tok_full = jnp.tile(masked_token.reshape(16, 48), (G, G))
    rrows = (jnp.arange(HH, dtype=jnp.int32)[:, None] // 16
             == jnp.arange(G, dtype=jnp.int32)[None, :]).astype(jnp.bfloat16)
    rcols = (jnp.arange(ROWW, dtype=jnp.int32)[None, :] // 48
             == jnp.arange(G, dtype=jnp.int32)[:, None]).astype(jnp.bfloat16)

    x3 = x.reshape(N, HH, ROWW)
    out3 = pl.pallas_call(
        _tc_select_body,
        grid=(N, HH // HB),
        in_specs=[
            pl.BlockSpec((1, G, G), lambda b, h: (b, 0, 0)),
            pl.BlockSpec((1, HB, ROWW), lambda b, h: (b, h, 0)),
            pl.BlockSpec((HB, ROWW), lambda b, h: (h, 0)),
            pl.BlockSpec((HB, G), lambda b, h: (h, 0)),
            pl.BlockSpec((G, ROWW), lambda b, h: (0, 0)),
        ],
        out_specs=pl.BlockSpec((1, HB, ROWW), lambda b, h: (b, h, 0)),
        out_shape=jax.ShapeDtypeStruct((N, HH, ROWW), jnp.float32),
    )(mask, x3, tok_full, rrows, rcols)
    return out3.reshape(N, HH, HH, 3)


# speedup vs baseline: 1.4302x; 1.0429x over previous
import jax
import jax.numpy as jnp
from jax import lax
from jax.experimental import pallas as pl
from jax.experimental.pallas import tpu as pltpu
from jax.experimental.pallas import tpu_sc as plsc

N = 64
HH = 512
ROWW = 1536
G = 32
NP = G * G
NUM_MASKED = 768
NC, NS = 2, 16
NW = NC * NS


def _sc_mask_body(idx_hbm, mask_hbm, idx_v, mask_v):
    wid = lax.axis_index("s") * NC + lax.axis_index("c")
    zeros16 = jnp.zeros((16,), jnp.int32)
    ones16 = jnp.ones((16,), jnp.int32)

    for s in range(2):
        b = 2 * wid + s
        pltpu.sync_copy(
            idx_hbm.at[pl.ds(8 * (b // 8), 8),
                       pl.ds(NUM_MASKED, NP - NUM_MASKED)],
            idx_v)
        rb = b % 8
        for r in range(G):
            mask_v[r, pl.ds(0, 16)] = zeros16
            mask_v[r, pl.ds(16, 16)] = zeros16
        for j in range(16):
            iv = idx_v[rb, pl.ds(j * 16, 16)]
            plsc.store_scatter(mask_v, [iv >> 5, iv & (G - 1)], ones16)
        pltpu.sync_copy(mask_v, mask_hbm.at[b])


HB = 128


def _tc_select_body(mask_ref, x_ref, tok_ref, rrows_ref, rcols_ref, out_ref):
    u = mask_ref[0].astype(jnp.bfloat16)
    mrows = jax.lax.dot_general(
        rrows_ref[...], u, (((1,), (0,)), ((), ())),
        preferred_element_type=jnp.float32)
    m = jax.lax.dot_general(
        mrows.astype(jnp.bfloat16), rcols_ref[...], (((1,), (0,)), ((), ())),
        preferred_element_type=jnp.float32)
    out_ref[0] = jnp.where(m > 0.5, x_ref[0], tok_ref[...])


def kernel(x, masked_token, shuffle_indices):
    idx2 = shuffle_indices.astype(jnp.int32)

    sc_mesh = plsc.VectorSubcoreMesh(core_axis_name="c", subcore_axis_name="s",
                                     num_cores=NC, num_subcores=NS)
    sc_mask = pl.kernel(
        _sc_mask_body,
        out_type=jax.ShapeDtypeStruct((N, G, G), jnp.int32),
        mesh=sc_mesh,
        compiler_params=pltpu.CompilerParams(needs_layout_passes=False),
        scratch_types=[
            pltpu.VMEM((8, NP - NUM_MASKED), jnp.int32),
            pltpu.VMEM((G, G), jnp.int32),
        ],
    )
    mask = sc_mask(idx2)

    tok_full = jnp.tile(masked_token.reshape(16, 48), (G, G))
    rrows = (jnp.arange(HH, dtype=jnp.int32)[:, None] // 16
             == jnp.arange(G, dtype=jnp.int32)[None, :]).astype(jnp.bfloat16)
    rcols = (jnp.arange(ROWW, dtype=jnp.int32)[None, :] // 48
             == jnp.arange(G, dtype=jnp.int32)[:, None]).astype(jnp.bfloat16)

    x3 = x.reshape(N, HH, ROWW)
    out3 = pl.pallas_call(
        _tc_select_body,
        grid=(HH // HB, N),
        in_specs=[
            pl.BlockSpec((1, G, G), lambda h, b: (b, 0, 0)),
            pl.BlockSpec((1, HB, ROWW), lambda h, b: (b, h, 0)),
            pl.BlockSpec((HB, ROWW), lambda h, b: (h, 0)),
            pl.BlockSpec((HB, G), lambda h, b: (h, 0)),
            pl.BlockSpec((G, ROWW), lambda h, b: (0, 0)),
        ],
        out_specs=pl.BlockSpec((1, HB, ROWW), lambda h, b: (b, h, 0)),
        out_shape=jax.ShapeDtypeStruct((N, HH, ROWW), jnp.float32),
    )(mask, x3, tok_full, rrows, rcols)
    return out3.reshape(N, HH, HH, 3)
